# Initial kernel scaffold; baseline (speedup 1.0000x reference)
import jax
import jax.numpy as jnp
from jax.experimental import pallas as pl


def kernel(u, v, boundary_norm, is_boundary, y_force, pos, params, edge_index):
    raise NotImplementedError("write your pallas kernel here")


# speedup vs baseline: 6.5724x; 6.5724x over previous
import functools

import numpy as np
import jax
import jax.numpy as jnp
from jax import lax
from jax.experimental import pallas as pl
from jax.experimental.pallas import tpu as pltpu
from jax.experimental.pallas import tpu_sc as plsc

L = 7
NR = 16
NS = 32
RD = 16
DIM = 256
NA = NR * L
N_NODES = 10000
N_EDGES = 160000

NC = 2
NT = 16
NW = NC * NT

KG = 40

BE = 1000
BM = 640
NP = 10240
BN = 640

_ACT_SLOPE = 0.01


def _act(x):
    return jnp.where(x >= 0, x, _ACT_SLOPE * x)


_MESH = plsc.VectorSubcoreMesh(core_axis_name="c", subcore_axis_name="s")


@functools.partial(jax.jit, static_argnames=("dcols",))
def _sc_gather(table, idx, dcols):
    M = idx.shape[0]
    per_tile = M // NW
    nchunk = per_tile // KG

    @functools.partial(
        pl.kernel,
        mesh=_MESH,
        compiler_params=pltpu.CompilerParams(needs_layout_passes=False),
        out_type=jax.ShapeDtypeStruct((M, dcols), jnp.float32),
        scratch_types=[
            pltpu.VMEM((KG,), jnp.int32),
            pltpu.VMEM((KG, dcols), jnp.float32),
            pltpu.SemaphoreType.DMA,
        ],
    )
    def k(table_hbm, idx_hbm, out_hbm, idx_v, rows_v, sem):
        wid = lax.axis_index("s") * NC + lax.axis_index("c")
        base = wid * per_tile

        def body(i, carry):
            off = base + i * KG
            pltpu.sync_copy(idx_hbm.at[pl.ds(off, KG)], idx_v)
            pltpu.async_copy(table_hbm.at[idx_v], rows_v, sem).wait()
            pltpu.sync_copy(rows_v, out_hbm.at[pl.ds(off, KG)])
            return carry

        lax.fori_loop(0, nchunk, body, 0)

    return k(table, idx)


NHH = 5120
NHD = 5128
KSC = 128
KSD = 80


@jax.jit
def _sc_scatter_add(mt, idx, zeros_blk):
    nchunk = N_EDGES // KSC

    @functools.partial(
        pl.kernel,
        mesh=_MESH,
        compiler_params=pltpu.CompilerParams(needs_layout_passes=False),
        out_type=jax.ShapeDtypeStruct((DIM, NP), jnp.float32),
        scratch_types=[
            pltpu.VMEM((16, NHD), jnp.float32),
            pltpu.VMEM((KSC,), jnp.int32),
            pltpu.VMEM((16, KSC), jnp.float32),
        ],
    )
    def k(mt_hbm, idx_hbm, z_hbm, out_hbm, acc, idx_v, m_v):
        cid = lax.axis_index("c")
        sid = lax.axis_index("s")
        half = sid // 8
        cs = (sid % 8) * 2 + cid
        pltpu.sync_copy(z_hbm, acc)
        base = jnp.full((16,), half * NHH, jnp.int32)
        iota = lax.broadcasted_iota(jnp.int32, (16,), 0)

        def body(i, carry):
            off = i * KSC
            pltpu.sync_copy(idx_hbm.at[pl.ds(off, KSC)], idx_v)
            pltpu.sync_copy(
                mt_hbm.at[pl.ds(cs * 16, 16), pl.ds(off, KSC)], m_v)
            for j in range(KSC // 16):
                iv = idx_v[pl.ds(j * 16, 16)]
                li = iv - base
                ok = (li >= 0) & (li < NHH)
                li = jnp.where(ok, li, NHH)
                cols = iota + (j * 16)
                for c in range(16):
                    cc = jnp.full((16,), c, jnp.int32)
                    vals = plsc.load_gather(m_v, [cc, cols])
                    plsc.addupdate_scatter(acc, [cc, li], vals)
            return carry

        lax.fori_loop(0, nchunk, body, 0)
        pltpu.sync_copy(
            acc.at[:, pl.ds(0, NHH)],
            out_hbm.at[pl.ds(cs * 16, 16), pl.ds(half * NHH, NHH)],
        )

    return k(mt, idx, zeros_blk)


@jax.jit
def _sc_degree(idx, zeros_blk):
    per_cs = N_EDGES // 16
    nchunk = per_cs // KSD

    @functools.partial(
        pl.kernel,
        mesh=_MESH,
        compiler_params=pltpu.CompilerParams(needs_layout_passes=False),
        out_type=jax.ShapeDtypeStruct((DIM, NP), jnp.float32),
        scratch_types=[
            pltpu.VMEM((16, NHD), jnp.float32),
            pltpu.VMEM((KSD,), jnp.int32),
        ],
    )
    def k(idx_hbm, z_hbm, out_hbm, acc, idx_v):
        cid = lax.axis_index("c")
        sid = lax.axis_index("s")
        half = sid // 8
        cs = (sid % 8) * 2 + cid
        pltpu.sync_copy(z_hbm, acc)
        base = jnp.full((16,), half * NHH, jnp.int32)

        ones16 = jnp.ones((16,), jnp.float32)
        zrow = jnp.zeros((16,), jnp.int32)

        def body(i, carry):
            off = cs * per_cs + i * KSD
            pltpu.sync_copy(idx_hbm.at[pl.ds(off, KSD)], idx_v)
            for j in range(KSD // 16):
                iv = idx_v[pl.ds(j * 16, 16)]
                li = iv - base
                ok = (li >= 0) & (li < NHH)
                li = jnp.where(ok, li, NHH)
                plsc.addupdate_scatter(acc, [zrow, li], ones16)
            return carry

        lax.fori_loop(0, nchunk, body, 0)
        pltpu.sync_copy(
            acc.at[:, pl.ds(0, NHH)],
            out_hbm.at[pl.ds(cs * 16, 16), pl.ds(half * NHH, NHH)],
        )

    return k(idx, zeros_blk)


def _cheb(cx, sy):
    cs, ss = [cx], [sy]
    for _ in range(L - 1):
        cs.append(cs[-1] * cx - ss[-1] * sy)
        ss.append(ss[-1] * cx + cs[-2] * sy)
    return cs, ss


def _edge_static_body(pr_ref, pc_ref, out_ref):
    ex = pr_ref[:, 0:1] - pc_ref[:, 0:1]
    ey = pr_ref[:, 1:2] - pc_ref[:, 1:2]
    r = jnp.sqrt(ex * ex + ey * ey)
    good = r > 0
    invr = jnp.where(good, 1.0 / jnp.where(good, r, 1.0), 0.0)
    c1 = ex * invr + (~good).astype(jnp.float32)
    s1 = ey * invr
    cs, ss = _cheb(c1, s1)
    xc = jnp.clip(r, 1e-6, None)
    nio = lax.broadcasted_iota(
        jnp.int32, (out_ref.shape[0], RD), 1).astype(jnp.float32) + 1.0
    de = jnp.sin(nio * (np.pi * xc)) / xc
    pad = jnp.zeros_like(r)
    out_ref[...] = jnp.concatenate(cs + ss + [r, pad] + [de], axis=1)


def _node_init_body(in_ref, wvt_ref, ws1_ref, bs1_ref, e7_ref, r7_ref,
                    xf_ref, cth_ref):
    u = in_ref[:, 0:4]
    vx = in_ref[:, 4:8]
    vy = in_ref[:, 8:12]
    bn = in_ref[:, 12:14]
    ib = in_ref[:, 14:15]
    yf = in_ref[:, 15:17]
    px = in_ref[:, 17:18]
    py = in_ref[:, 18:19]

    r = jnp.sqrt(px * px + py * py)
    good = r > 0
    invr = jnp.where(good, 1.0 / jnp.where(good, r, 1.0), 0.0)
    c1 = px * invr + (~good).astype(jnp.float32)
    s1 = py * invr
    cs, ss = _cheb(c1, s1)
    ct = jnp.concatenate(cs, axis=1)
    st = jnp.concatenate(ss, axis=1)
    pad = jnp.zeros_like(c1)
    cth_ref[...] = jnp.concatenate([ct, st, pad, pad], axis=1)

    vax = jnp.concatenate([vx, bn[:, 0:1], yf[:, 0:1]], axis=1)
    vay = jnp.concatenate([vy, bn[:, 1:2], yf[:, 1:2]], axis=1)
    wx = jnp.dot(vax, wvt_ref[...], preferred_element_type=jnp.float32)
    wy = jnp.dot(vay, wvt_ref[...], preferred_element_type=jnp.float32)
    cf = jnp.dot(ct, e7_ref[...], preferred_element_type=jnp.float32)
    sf = jnp.dot(st, e7_ref[...], preferred_element_type=jnp.float32)
    wxf = jnp.dot(wx, r7_ref[...], preferred_element_type=jnp.float32)
    wyf = jnp.dot(wy, r7_ref[...], preferred_element_type=jnp.float32)
    a0 = cf * wxf - sf * wyf
    b0 = sf * wxf + cf * wyf

    vn = jnp.sqrt(vx * vx + vy * vy)
    bnn = jnp.sqrt(bn[:, 0:1] ** 2 + bn[:, 1:2] ** 2)
    yfn = jnp.sqrt(yf[:, 0:1] ** 2 + yf[:, 1:2] ** 2)
    s_in = jnp.concatenate([u, vn, ib, bnn, yfn], axis=1)
    xs = _act(jnp.dot(s_in, ws1_ref[...], preferred_element_type=jnp.float32)
              + bs1_ref[...])
    xf_ref[...] = jnp.concatenate([xs, a0, b0], axis=1)


def _edge_mlp_body(g_ref, es_ref, w1_ref, b1_ref, w2_ref, b2_ref, e7_ref,
                   m_ref):
    es = es_ref[...]
    cf = jnp.dot(es[:, 0:L], e7_ref[...], preferred_element_type=jnp.float32)
    sf = jnp.dot(es[:, L:2 * L], e7_ref[...],
                 preferred_element_type=jnp.float32)
    g = g_ref[...]
    gs = g[:, :NS]
    ga = g[:, NS:NS + NA]
    gb = g[:, NS + NA:]
    fa = cf * ga - sf * gb
    fb = sf * ga + cf * gb
    m_in = jnp.concatenate([gs, fa, fb, es[:, 16:32]], axis=1)
    h = _act(jnp.dot(m_in, w1_ref[...], preferred_element_type=jnp.float32)
             + b1_ref[...])
    m = jnp.dot(h, w2_ref[...], preferred_element_type=jnp.float32) + b2_ref[...]
    ms = m[:, :NS]
    ma = m[:, NS:NS + NA]
    mb = m[:, NS + NA:]
    mm = jnp.concatenate(
        [ms, ma * cf + mb * sf, mb * cf - ma * sf], axis=1)
    m_ref[...] = mm.T


def _node_update_body(xf_ref, agg_ref, deg_ref, wus_ref, bus_ref, kur_ref,
                      out_ref):
    deg = jnp.sum(deg_ref[...], axis=0, keepdims=True)
    inv = 1.0 / jnp.maximum(deg, 1.0)
    agg = (agg_ref[...] * inv).T
    xf = xf_ref[...]
    xs = xf[:, :NS]
    xa = xf[:, NS:NS + NA]
    xb = xf[:, NS + NA:]
    ags = agg[:, :NS]
    aga = agg[:, NS:NS + NA]
    agb = agg[:, NS + NA:]
    xs2 = xs + _act(jnp.dot(ags, wus_ref[...],
                            preferred_element_type=jnp.float32) + bus_ref[...])
    xa2 = xa + jnp.dot(aga, kur_ref[...], preferred_element_type=jnp.float32)
    xb2 = xb + jnp.dot(agb, kur_ref[...], preferred_element_type=jnp.float32)
    out_ref[...] = jnp.concatenate([xs2, xa2, xb2], axis=1)


def _head_body(xf_ref, cth_ref, ul_ref, lng_ref, lnb_ref, lngr_ref,
               e7_ref, r7_ref, r7t_ref, wv1_ref, bv1_ref, wv2_ref, bv2_ref,
               wu1_ref, bu1_ref, wu2_ref, bu2_ref, o1_ref, o2_ref):
    xf = xf_ref[...]
    xs = xf[:, :NS]
    xa = xf[:, NS:NS + NA]
    xb = xf[:, NS + NA:]
    mu = jnp.mean(xs, axis=1, keepdims=True)
    xc = xs - mu
    var = jnp.mean(xc * xc, axis=1, keepdims=True)
    xsn = lng_ref[...] * xc * lax.rsqrt(var + 1e-5) + lnb_ref[...]

    rsum = jnp.dot(xa * xa + xb * xb, r7t_ref[...],
                   preferred_element_type=jnp.float32)
    gor = lngr_ref[...] * lax.rsqrt(rsum / (2.0 * L) + 1e-5)
    gof = jnp.dot(gor, r7_ref[...], preferred_element_type=jnp.float32)
    xa2 = xa * gof
    xb2 = xb * gof

    cth = cth_ref[...]
    cf = jnp.dot(cth[:, 0:L], e7_ref[...], preferred_element_type=jnp.float32)
    sf = jnp.dot(cth[:, L:2 * L], e7_ref[...],
                 preferred_element_type=jnp.float32)
    oa = cf * xa2 - sf * xb2
    ob = sf * xa2 + cf * xb2
    feat = jnp.concatenate([xsn, oa, ob], axis=1)

    hv = _act(jnp.dot(feat, wv1_ref[...], preferred_element_type=jnp.float32)
              + bv1_ref[...])
    ov = jnp.dot(hv, wv2_ref[...], preferred_element_type=jnp.float32) + bv2_ref[...]
    hu = _act(jnp.dot(feat, wu1_ref[...], preferred_element_type=jnp.float32)
              + bu1_ref[...])
    ou = jnp.dot(hu, wu2_ref[...], preferred_element_type=jnp.float32) + bu2_ref[...]

    ul = ul_ref[...]
    c1 = cth[:, 0:1]
    s1 = cth[:, L:L + 1]
    z6 = jnp.zeros((ov.shape[0], 6), jnp.float32)
    v0 = ov[:, 0:1] * c1 + ov[:, 1:2] * s1 + ul[:, 1:2]
    v1 = ov[:, 1:2] * c1 - ov[:, 0:1] * s1 + ul[:, 2:3]
    o2_ref[...] = jnp.concatenate([v0, v1, z6], axis=1)
    o1 = ou[:, 0:1] + ul[:, 0:1]
    z7 = jnp.zeros((ov.shape[0], 7), jnp.float32)
    o1_ref[...] = jnp.concatenate([o1, z7], axis=1)


def _full(shape):
    return pl.BlockSpec(shape, lambda i: (0, 0))


def _blk(bs, width):
    return pl.BlockSpec((bs, width), lambda i: (i, 0))


def kernel(u, v, boundary_norm, is_boundary, y_force, pos, params, edge_index):
    N = N_NODES
    E = N_EDGES
    f32 = jnp.float32

    posc = pos - pos.mean(axis=0)
    row = edge_index[0]
    col = edge_index[1]

    E7 = jnp.tile(jnp.eye(L, dtype=f32), (1, NR))
    R7 = jnp.repeat(jnp.eye(NR, dtype=f32), L, axis=1)
    R7T = R7.T

    layer_ws = []
    for b in params['blocks']:
        W1 = b['W1']
        W1r = W1[NS:NS + 2 * NA].reshape(NR, L, 2, DIM)
        W1p = jnp.concatenate([W1[:NS], W1r[:, :, 0].reshape(NA, DIM),
                               W1r[:, :, 1].reshape(NA, DIM),
                               W1[NS + 2 * NA:]], axis=0)
        W2 = b['W2']
        W2r = W2[:, NS:].reshape(DIM, NR, L, 2)
        W2p = jnp.concatenate([W2[:, :NS], W2r[..., 0].reshape(DIM, NA),
                               W2r[..., 1].reshape(DIM, NA)], axis=1)
        b2 = b['b2']
        b2r = b2[NS:].reshape(NR, L, 2)
        b2p = jnp.concatenate([b2[:NS], b2r[:, :, 0].ravel(),
                               b2r[:, :, 1].ravel()])
        Kur = jnp.kron(b['W_ur'].T, jnp.eye(L, dtype=f32))
        layer_ws.append((W1p, b['b1'][None, :], W2p, b2p[None, :],
                         b['W_us'], b['b_us'][None, :], Kur))

    pa = np.array([NS + r * 2 * L + k * 2 for r in range(NR) for k in range(L)])
    Wv1p = jnp.concatenate([params['Wv1'][:NS], params['Wv1'][pa],
                            params['Wv1'][pa + 1]], axis=0)
    Wu1p = jnp.concatenate([params['Wu1'][:NS], params['Wu1'][pa],
                            params['Wu1'][pa + 1]], axis=0)
    Wv2p = jnp.zeros((DIM * 3, 8), f32).at[:, :2].set(params['Wv2'])
    bv2p = jnp.zeros((1, 8), f32).at[0, :2].set(params['bv2'])
    Wu2p = jnp.zeros((DIM * 3, 8), f32).at[:, :1].set(params['Wu2'])
    bu2p = jnp.zeros((1, 8), f32).at[0, :1].set(params['bu2'])

    p128 = jnp.zeros((N, 128), f32).at[:, :2].set(posc)
    pe = _sc_gather(p128, jnp.concatenate([row, col]), 128)

    zacc = jnp.zeros((16, NHD), f32)
    degf = _sc_degree(row, zacc)

    ge = E // BE
    es = pl.pallas_call(
        _edge_static_body,
        grid=(ge,),
        in_specs=[pl.BlockSpec((BE, 128), lambda i: (i, 0)),
                  pl.BlockSpec((BE, 128), lambda i: (i + ge, 0))],
        out_specs=_blk(BE, 32),
        out_shape=jax.ShapeDtypeStruct((E, 32), f32),
    )(pe, pe)

    node_in = jnp.zeros((NP, 32), f32).at[:N, :19].set(jnp.concatenate(
        [u, v[:, :, 0], v[:, :, 1], boundary_norm, is_boundary, y_force,
         posc], axis=1))
    gn = NP // BN
    xf, cth = pl.pallas_call(
        _node_init_body,
        grid=(gn,),
        in_specs=[_blk(BN, 32), _full((6, NR)), _full((11, NS)),
                  _full((1, NS)), _full((L, NA)), _full((NR, NA))],
        out_specs=[_blk(BN, DIM), _blk(BN, 16)],
        out_shape=[jax.ShapeDtypeStruct((NP, DIM), f32),
                   jax.ShapeDtypeStruct((NP, 16), f32)],
    )(node_in, params['W_vec'].T, params['W_s1'], params['b_s1'][None, :],
      E7, R7)

    gm = E // BM
    for (W1p, b1, W2p, b2p, Wus, bus, Kur) in layer_ws:
        g = _sc_gather(xf, col, DIM)
        mt = pl.pallas_call(
            _edge_mlp_body,
            grid=(gm,),
            in_specs=[_blk(BM, DIM), _blk(BM, 32), _full((DIM + 16, DIM)),
                      _full((1, DIM)), _full((DIM, DIM)), _full((1, DIM)),
                      _full((L, NA))],
            out_specs=pl.BlockSpec((DIM, BM), lambda i: (0, i)),
            out_shape=jax.ShapeDtypeStruct((DIM, E), f32),
        )(g, es, W1p, b1, W2p, b2p, E7)
        agg = _sc_scatter_add(mt, row, zacc)
        xf = pl.pallas_call(
            _node_update_body,
            grid=(gn,),
            in_specs=[_blk(BN, DIM),
                      pl.BlockSpec((DIM, BN), lambda i: (0, i)),
                      pl.BlockSpec((DIM, BN), lambda i: (0, i)),
                      _full((NS, NS)), _full((1, NS)), _full((NA, NA))],
            out_specs=_blk(BN, DIM),
            out_shape=jax.ShapeDtypeStruct((NP, DIM), f32),
        )(xf, agg, degf, Wus, bus, Kur)

    ul = jnp.zeros((NP, 16), f32)
    ul = ul.at[:N, 0].set(u[:, -1]).at[:N, 1:3].set(v[:, -1, :])
    o1, o2 = pl.pallas_call(
        _head_body,
        grid=(gn,),
        in_specs=[_blk(BN, DIM), _blk(BN, 16), _blk(BN, 16),
                  _full((1, NS)), _full((1, NS)), _full((1, NR)),
                  _full((L, NA)), _full((NR, NA)), _full((NA, NR)),
                  _full((DIM, DIM * 3)), _full((1, DIM * 3)),
                  _full((DIM * 3, 8)), _full((1, 8)),
                  _full((DIM, DIM * 3)), _full((1, DIM * 3)),
                  _full((DIM * 3, 8)), _full((1, 8))],
        out_specs=[_blk(BN, 8), _blk(BN, 8)],
        out_shape=[jax.ShapeDtypeStruct((NP, 8), f32),
                   jax.ShapeDtypeStruct((NP, 8), f32)],
    )(xf, cth, ul, params['ln_g'][None, :], params['ln_b'][None, :],
      params['ln_gr'][None, :], E7, R7, R7T, Wv1p, params['bv1'][None, :],
      Wv2p, bv2p, Wu1p, params['bu1'][None, :], Wu2p, bu2p)

    return (o1[:N, 0], o2[:N, :2])


# speedup vs baseline: 6.6104x; 1.0058x over previous
import functools

import numpy as np
import jax
import jax.numpy as jnp
from jax import lax
from jax.experimental import pallas as pl
from jax.experimental.pallas import tpu as pltpu
from jax.experimental.pallas import tpu_sc as plsc

L = 7
NR = 16
NS = 32
RD = 16
DIM = 256
NA = NR * L
N_NODES = 10000
N_EDGES = 160000

NC = 2
NT = 16
NW = NC * NT

KG = 40

BE = 1000
BM = 640
NP = 10240
BN = 640

_ACT_SLOPE = 0.01


def _act(x):
    return jnp.where(x >= 0, x, _ACT_SLOPE * x)


_MESH = plsc.VectorSubcoreMesh(core_axis_name="c", subcore_axis_name="s")


@functools.partial(jax.jit, static_argnames=("dcols",))
def _sc_gather(table, idx, dcols):
    M = idx.shape[0]
    per_tile = M // NW
    nchunk = per_tile // KG

    @functools.partial(
        pl.kernel,
        mesh=_MESH,
        compiler_params=pltpu.CompilerParams(needs_layout_passes=False),
        out_type=jax.ShapeDtypeStruct((M, dcols), jnp.float32),
        scratch_types=[
            pltpu.VMEM((KG,), jnp.int32),
            pltpu.VMEM((KG, dcols), jnp.float32),
            pltpu.SemaphoreType.DMA,
        ],
    )
    def k(table_hbm, idx_hbm, out_hbm, idx_v, rows_v, sem):
        wid = lax.axis_index("s") * NC + lax.axis_index("c")
        base = wid * per_tile

        def body(i, carry):
            off = base + i * KG
            pltpu.sync_copy(idx_hbm.at[pl.ds(off, KG)], idx_v)
            pltpu.async_copy(table_hbm.at[idx_v], rows_v, sem).wait()
            pltpu.sync_copy(rows_v, out_hbm.at[pl.ds(off, KG)])
            return carry

        lax.fori_loop(0, nchunk, body, 0)

    return k(table, idx)


NHH = 5120
NHD = 5128
KSC = 128
KSD = 80


@jax.jit
def _sc_scatter_add(mt, idx, zeros_blk):
    nchunk = N_EDGES // KSC

    @functools.partial(
        pl.kernel,
        mesh=_MESH,
        compiler_params=pltpu.CompilerParams(needs_layout_passes=False),
        out_type=jax.ShapeDtypeStruct((DIM, NP), jnp.float32),
        scratch_types=[
            pltpu.VMEM((16, NHD), jnp.float32),
            pltpu.VMEM((KSC,), jnp.int32),
            pltpu.VMEM((16, KSC), jnp.float32),
        ],
    )
    def k(mt_hbm, idx_hbm, z_hbm, out_hbm, acc, idx_v, m_v):
        cid = lax.axis_index("c")
        sid = lax.axis_index("s")
        half = sid // 8
        cs = (sid % 8) * 2 + cid
        pltpu.sync_copy(z_hbm, acc)
        base = jnp.full((16,), half * NHH, jnp.int32)
        iota = lax.broadcasted_iota(jnp.int32, (16,), 0)

        def body(i, carry):
            off = i * KSC
            pltpu.sync_copy(idx_hbm.at[pl.ds(off, KSC)], idx_v)
            pltpu.sync_copy(
                mt_hbm.at[pl.ds(cs * 16, 16), pl.ds(off, KSC)], m_v)
            for j in range(KSC // 16):
                iv = idx_v[pl.ds(j * 16, 16)]
                li = iv - base
                ok = (li >= 0) & (li < NHH)
                li = jnp.where(ok, li, NHH)
                for c in range(16):
                    cc = jnp.full((16,), c, jnp.int32)
                    vals = m_v[c, pl.ds(j * 16, 16)]
                    plsc.addupdate_scatter(acc, [cc, li], vals)
            return carry

        lax.fori_loop(0, nchunk, body, 0)
        pltpu.sync_copy(
            acc.at[:, pl.ds(0, NHH)],
            out_hbm.at[pl.ds(cs * 16, 16), pl.ds(half * NHH, NHH)],
        )

    return k(mt, idx, zeros_blk)


@jax.jit
def _sc_degree(idx, zeros_blk):
    per_cs = N_EDGES // 16
    nchunk = per_cs // KSD

    @functools.partial(
        pl.kernel,
        mesh=_MESH,
        compiler_params=pltpu.CompilerParams(needs_layout_passes=False),
        out_type=jax.ShapeDtypeStruct((DIM, NP), jnp.float32),
        scratch_types=[
            pltpu.VMEM((16, NHD), jnp.float32),
            pltpu.VMEM((KSD,), jnp.int32),
        ],
    )
    def k(idx_hbm, z_hbm, out_hbm, acc, idx_v):
        cid = lax.axis_index("c")
        sid = lax.axis_index("s")
        half = sid // 8
        cs = (sid % 8) * 2 + cid
        pltpu.sync_copy(z_hbm, acc)
        base = jnp.full((16,), half * NHH, jnp.int32)

        ones16 = jnp.ones((16,), jnp.float32)
        zrow = jnp.zeros((16,), jnp.int32)

        def body(i, carry):
            off = cs * per_cs + i * KSD
            pltpu.sync_copy(idx_hbm.at[pl.ds(off, KSD)], idx_v)
            for j in range(KSD // 16):
                iv = idx_v[pl.ds(j * 16, 16)]
                li = iv - base
                ok = (li >= 0) & (li < NHH)
                li = jnp.where(ok, li, NHH)
                plsc.addupdate_scatter(acc, [zrow, li], ones16)
            return carry

        lax.fori_loop(0, nchunk, body, 0)
        pltpu.sync_copy(
            acc.at[:, pl.ds(0, NHH)],
            out_hbm.at[pl.ds(cs * 16, 16), pl.ds(half * NHH, NHH)],
        )

    return k(idx, zeros_blk)


def _cheb(cx, sy):
    cs, ss = [cx], [sy]
    for _ in range(L - 1):
        cs.append(cs[-1] * cx - ss[-1] * sy)
        ss.append(ss[-1] * cx + cs[-2] * sy)
    return cs, ss


def _edge_static_body(pr_ref, pc_ref, out_ref):
    ex = pr_ref[:, 0:1] - pc_ref[:, 0:1]
    ey = pr_ref[:, 1:2] - pc_ref[:, 1:2]
    r = jnp.sqrt(ex * ex + ey * ey)
    good = r > 0
    invr = jnp.where(good, 1.0 / jnp.where(good, r, 1.0), 0.0)
    c1 = ex * invr + (~good).astype(jnp.float32)
    s1 = ey * invr
    cs, ss = _cheb(c1, s1)
    xc = jnp.clip(r, 1e-6, None)
    nio = lax.broadcasted_iota(
        jnp.int32, (out_ref.shape[0], RD), 1).astype(jnp.float32) + 1.0
    de = jnp.sin(nio * (np.pi * xc)) / xc
    pad = jnp.zeros_like(r)
    out_ref[...] = jnp.concatenate(cs + ss + [r, pad] + [de], axis=1)


def _node_init_body(in_ref, wvt_ref, ws1_ref, bs1_ref, e7_ref, r7_ref,
                    xf_ref, cth_ref):
    u = in_ref[:, 0:4]
    vx = in_ref[:, 4:8]
    vy = in_ref[:, 8:12]
    bn = in_ref[:, 12:14]
    ib = in_ref[:, 14:15]
    yf = in_ref[:, 15:17]
    px = in_ref[:, 17:18]
    py = in_ref[:, 18:19]

    r = jnp.sqrt(px * px + py * py)
    good = r > 0
    invr = jnp.where(good, 1.0 / jnp.where(good, r, 1.0), 0.0)
    c1 = px * invr + (~good).astype(jnp.float32)
    s1 = py * invr
    cs, ss = _cheb(c1, s1)
    ct = jnp.concatenate(cs, axis=1)
    st = jnp.concatenate(ss, axis=1)
    pad = jnp.zeros_like(c1)
    cth_ref[...] = jnp.concatenate([ct, st, pad, pad], axis=1)

    vax = jnp.concatenate([vx, bn[:, 0:1], yf[:, 0:1]], axis=1)
    vay = jnp.concatenate([vy, bn[:, 1:2], yf[:, 1:2]], axis=1)
    wx = jnp.dot(vax, wvt_ref[...], preferred_element_type=jnp.float32)
    wy = jnp.dot(vay, wvt_ref[...], preferred_element_type=jnp.float32)
    cf = jnp.dot(ct, e7_ref[...], preferred_element_type=jnp.float32)
    sf = jnp.dot(st, e7_ref[...], preferred_element_type=jnp.float32)
    wxf = jnp.dot(wx, r7_ref[...], preferred_element_type=jnp.float32)
    wyf = jnp.dot(wy, r7_ref[...], preferred_element_type=jnp.float32)
    a0 = cf * wxf - sf * wyf
    b0 = sf * wxf + cf * wyf

    vn = jnp.sqrt(vx * vx + vy * vy)
    bnn = jnp.sqrt(bn[:, 0:1] ** 2 + bn[:, 1:2] ** 2)
    yfn = jnp.sqrt(yf[:, 0:1] ** 2 + yf[:, 1:2] ** 2)
    s_in = jnp.concatenate([u, vn, ib, bnn, yfn], axis=1)
    xs = _act(jnp.dot(s_in, ws1_ref[...], preferred_element_type=jnp.float32)
              + bs1_ref[...])
    xf_ref[...] = jnp.concatenate([xs, a0, b0], axis=1)


def _edge_mlp_body(g_ref, es_ref, w1_ref, b1_ref, w2_ref, b2_ref, e7_ref,
                   m_ref):
    es = es_ref[...]
    cf = jnp.dot(es[:, 0:L], e7_ref[...], preferred_element_type=jnp.float32)
    sf = jnp.dot(es[:, L:2 * L], e7_ref[...],
                 preferred_element_type=jnp.float32)
    g = g_ref[...]
    gs = g[:, :NS]
    ga = g[:, NS:NS + NA]
    gb = g[:, NS + NA:]
    fa = cf * ga - sf * gb
    fb = sf * ga + cf * gb
    m_in = jnp.concatenate([gs, fa, fb, es[:, 16:32]], axis=1)
    h = _act(jnp.dot(m_in, w1_ref[...], preferred_element_type=jnp.float32)
             + b1_ref[...])
    m = jnp.dot(h, w2_ref[...], preferred_element_type=jnp.float32) + b2_ref[...]
    ms = m[:, :NS]
    ma = m[:, NS:NS + NA]
    mb = m[:, NS + NA:]
    mm = jnp.concatenate(
        [ms, ma * cf + mb * sf, mb * cf - ma * sf], axis=1)
    m_ref[...] = mm.T


def _node_update_body(xf_ref, agg_ref, deg_ref, wus_ref, bus_ref, kur_ref,
                      out_ref):
    deg = jnp.sum(deg_ref[...], axis=0, keepdims=True)
    inv = 1.0 / jnp.maximum(deg, 1.0)
    agg = (agg_ref[...] * inv).T
    xf = xf_ref[...]
    xs = xf[:, :NS]
    xa = xf[:, NS:NS + NA]
    xb = xf[:, NS + NA:]
    ags = agg[:, :NS]
    aga = agg[:, NS:NS + NA]
    agb = agg[:, NS + NA:]
    xs2 = xs + _act(jnp.dot(ags, wus_ref[...],
                            preferred_element_type=jnp.float32) + bus_ref[...])
    xa2 = xa + jnp.dot(aga, kur_ref[...], preferred_element_type=jnp.float32)
    xb2 = xb + jnp.dot(agb, kur_ref[...], preferred_element_type=jnp.float32)
    out_ref[...] = jnp.concatenate([xs2, xa2, xb2], axis=1)


def _head_body(xf_ref, cth_ref, ul_ref, lng_ref, lnb_ref, lngr_ref,
               e7_ref, r7_ref, r7t_ref, wv1_ref, bv1_ref, wv2_ref, bv2_ref,
               wu1_ref, bu1_ref, wu2_ref, bu2_ref, o1_ref, o2_ref):
    xf = xf_ref[...]
    xs = xf[:, :NS]
    xa = xf[:, NS:NS + NA]
    xb = xf[:, NS + NA:]
    mu = jnp.mean(xs, axis=1, keepdims=True)
    xc = xs - mu
    var = jnp.mean(xc * xc, axis=1, keepdims=True)
    xsn = lng_ref[...] * xc * lax.rsqrt(var + 1e-5) + lnb_ref[...]

    rsum = jnp.dot(xa * xa + xb * xb, r7t_ref[...],
                   preferred_element_type=jnp.float32)
    gor = lngr_ref[...] * lax.rsqrt(rsum / (2.0 * L) + 1e-5)
    gof = jnp.dot(gor, r7_ref[...], preferred_element_type=jnp.float32)
    xa2 = xa * gof
    xb2 = xb * gof

    cth = cth_ref[...]
    cf = jnp.dot(cth[:, 0:L], e7_ref[...], preferred_element_type=jnp.float32)
    sf = jnp.dot(cth[:, L:2 * L], e7_ref[...],
                 preferred_element_type=jnp.float32)
    oa = cf * xa2 - sf * xb2
    ob = sf * xa2 + cf * xb2
    feat = jnp.concatenate([xsn, oa, ob], axis=1)

    hv = _act(jnp.dot(feat, wv1_ref[...], preferred_element_type=jnp.float32)
              + bv1_ref[...])
    ov = jnp.dot(hv, wv2_ref[...], preferred_element_type=jnp.float32) + bv2_ref[...]
    hu = _act(jnp.dot(feat, wu1_ref[...], preferred_element_type=jnp.float32)
              + bu1_ref[...])
    ou = jnp.dot(hu, wu2_ref[...], preferred_element_type=jnp.float32) + bu2_ref[...]

    ul = ul_ref[...]
    c1 = cth[:, 0:1]
    s1 = cth[:, L:L + 1]
    z6 = jnp.zeros((ov.shape[0], 6), jnp.float32)
    v0 = ov[:, 0:1] * c1 + ov[:, 1:2] * s1 + ul[:, 1:2]
    v1 = ov[:, 1:2] * c1 - ov[:, 0:1] * s1 + ul[:, 2:3]
    o2_ref[...] = jnp.concatenate([v0, v1, z6], axis=1)
    o1 = ou[:, 0:1] + ul[:, 0:1]
    z7 = jnp.zeros((ov.shape[0], 7), jnp.float32)
    o1_ref[...] = jnp.concatenate([o1, z7], axis=1)


def _full(shape):
    return pl.BlockSpec(shape, lambda i: (0, 0))


def _blk(bs, width):
    return pl.BlockSpec((bs, width), lambda i: (i, 0))


def kernel(u, v, boundary_norm, is_boundary, y_force, pos, params, edge_index):
    N = N_NODES
    E = N_EDGES
    f32 = jnp.float32

    posc = pos - pos.mean(axis=0)
    row = edge_index[0]
    col = edge_index[1]

    E7 = jnp.tile(jnp.eye(L, dtype=f32), (1, NR))
    R7 = jnp.repeat(jnp.eye(NR, dtype=f32), L, axis=1)
    R7T = R7.T

    layer_ws = []
    for b in params['blocks']:
        W1 = b['W1']
        W1r = W1[NS:NS + 2 * NA].reshape(NR, L, 2, DIM)
        W1p = jnp.concatenate([W1[:NS], W1r[:, :, 0].reshape(NA, DIM),
                               W1r[:, :, 1].reshape(NA, DIM),
                               W1[NS + 2 * NA:]], axis=0)
        W2 = b['W2']
        W2r = W2[:, NS:].reshape(DIM, NR, L, 2)
        W2p = jnp.concatenate([W2[:, :NS], W2r[..., 0].reshape(DIM, NA),
                               W2r[..., 1].reshape(DIM, NA)], axis=1)
        b2 = b['b2']
        b2r = b2[NS:].reshape(NR, L, 2)
        b2p = jnp.concatenate([b2[:NS], b2r[:, :, 0].ravel(),
                               b2r[:, :, 1].ravel()])
        Kur = jnp.kron(b['W_ur'].T, jnp.eye(L, dtype=f32))
        layer_ws.append((W1p, b['b1'][None, :], W2p, b2p[None, :],
                         b['W_us'], b['b_us'][None, :], Kur))

    pa = np.array([NS + r * 2 * L + k * 2 for r in range(NR) for k in range(L)])
    Wv1p = jnp.concatenate([params['Wv1'][:NS], params['Wv1'][pa],
                            params['Wv1'][pa + 1]], axis=0)
    Wu1p = jnp.concatenate([params['Wu1'][:NS], params['Wu1'][pa],
                            params['Wu1'][pa + 1]], axis=0)
    Wv2p = jnp.zeros((DIM * 3, 8), f32).at[:, :2].set(params['Wv2'])
    bv2p = jnp.zeros((1, 8), f32).at[0, :2].set(params['bv2'])
    Wu2p = jnp.zeros((DIM * 3, 8), f32).at[:, :1].set(params['Wu2'])
    bu2p = jnp.zeros((1, 8), f32).at[0, :1].set(params['bu2'])

    p128 = jnp.zeros((N, 128), f32).at[:, :2].set(posc)
    pe = _sc_gather(p128, jnp.concatenate([row, col]), 128)

    zacc = jnp.zeros((16, NHD), f32)
    degf = _sc_degree(row, zacc)

    ge = E // BE
    es = pl.pallas_call(
        _edge_static_body,
        grid=(ge,),
        in_specs=[pl.BlockSpec((BE, 128), lambda i: (i, 0)),
                  pl.BlockSpec((BE, 128), lambda i: (i + ge, 0))],
        out_specs=_blk(BE, 32),
        out_shape=jax.ShapeDtypeStruct((E, 32), f32),
    )(pe, pe)

    node_in = jnp.zeros((NP, 32), f32).at[:N, :19].set(jnp.concatenate(
        [u, v[:, :, 0], v[:, :, 1], boundary_norm, is_boundary, y_force,
         posc], axis=1))
    gn = NP // BN
    xf, cth = pl.pallas_call(
        _node_init_body,
        grid=(gn,),
        in_specs=[_blk(BN, 32), _full((6, NR)), _full((11, NS)),
                  _full((1, NS)), _full((L, NA)), _full((NR, NA))],
        out_specs=[_blk(BN, DIM), _blk(BN, 16)],
        out_shape=[jax.ShapeDtypeStruct((NP, DIM), f32),
                   jax.ShapeDtypeStruct((NP, 16), f32)],
    )(node_in, params['W_vec'].T, params['W_s1'], params['b_s1'][None, :],
      E7, R7)

    gm = E // BM
    for (W1p, b1, W2p, b2p, Wus, bus, Kur) in layer_ws:
        g = _sc_gather(xf, col, DIM)
        mt = pl.pallas_call(
            _edge_mlp_body,
            grid=(gm,),
            in_specs=[_blk(BM, DIM), _blk(BM, 32), _full((DIM + 16, DIM)),
                      _full((1, DIM)), _full((DIM, DIM)), _full((1, DIM)),
                      _full((L, NA))],
            out_specs=pl.BlockSpec((DIM, BM), lambda i: (0, i)),
            out_shape=jax.ShapeDtypeStruct((DIM, E), f32),
        )(g, es, W1p, b1, W2p, b2p, E7)
        agg = _sc_scatter_add(mt, row, zacc)
        xf = pl.pallas_call(
            _node_update_body,
            grid=(gn,),
            in_specs=[_blk(BN, DIM),
                      pl.BlockSpec((DIM, BN), lambda i: (0, i)),
                      pl.BlockSpec((DIM, BN), lambda i: (0, i)),
                      _full((NS, NS)), _full((1, NS)), _full((NA, NA))],
            out_specs=_blk(BN, DIM),
            out_shape=jax.ShapeDtypeStruct((NP, DIM), f32),
        )(xf, agg, degf, Wus, bus, Kur)

    ul = jnp.zeros((NP, 16), f32)
    ul = ul.at[:N, 0].set(u[:, -1]).at[:N, 1:3].set(v[:, -1, :])
    o1, o2 = pl.pallas_call(
        _head_body,
        grid=(gn,),
        in_specs=[_blk(BN, DIM), _blk(BN, 16), _blk(BN, 16),
                  _full((1, NS)), _full((1, NS)), _full((1, NR)),
                  _full((L, NA)), _full((NR, NA)), _full((NA, NR)),
                  _full((DIM, DIM * 3)), _full((1, DIM * 3)),
                  _full((DIM * 3, 8)), _full((1, 8)),
                  _full((DIM, DIM * 3)), _full((1, DIM * 3)),
                  _full((DIM * 3, 8)), _full((1, 8))],
        out_specs=[_blk(BN, 8), _blk(BN, 8)],
        out_shape=[jax.ShapeDtypeStruct((NP, 8), f32),
                   jax.ShapeDtypeStruct((NP, 8), f32)],
    )(xf, cth, ul, params['ln_g'][None, :], params['ln_b'][None, :],
      params['ln_gr'][None, :], E7, R7, R7T, Wv1p, params['bv1'][None, :],
      Wv2p, bv2p, Wu1p, params['bu1'][None, :], Wu2p, bu2p)

    return (o1[:N, 0], o2[:N, :2])
